# baseline (speedup 1.0000x reference)
import jax
import jax.numpy as jnp
import numpy as np
from jax import lax
from jax.experimental import pallas as pl
from jax.experimental.pallas import tpu as pltpu
from jax.experimental.pallas import tpu_sc as plsc

B = 4
N = 16800
L = 16
NC = 2
NS = 16
WPB = 8
C = N // WPB
U = 4
NV = -(-C // L)
VAR0 = np.float32(0.1)
NEG_INF = np.float32(-np.inf)
IMAX = np.int32(2**31 - 1)

_MESH = plsc.VectorSubcoreMesh(
    core_axis_name="c", subcore_axis_name="s", num_cores=NC, num_subcores=NS
)


def _sc_body(conf_hbm, landms_hbm, priors_hbm, out_hbm,
             sbuf, mstage, istage, mload, iload, tmpf, tmpi, lrow, prow, obuf,
             sem_l, sem_p):
    c = lax.axis_index("c")
    s = lax.axis_index("s")
    g = s // WPB
    w = s % WPB
    b = 2 * c + g
    base = w * C

    pltpu.sync_copy(conf_hbm.at[b, pl.ds(base, C), :], sbuf.at[pl.ds(0, C), :])

    lane = lax.iota(jnp.int32, L)
    ones = jnp.ones((L,), jnp.int32)

    def scan_vec(j, carry):
        run_max, run_idx = carry
        n = j * L + lane
        v = plsc.load_gather(sbuf, [n, ones])
        v = jnp.where(v > 0.0, v, NEG_INF)
        upd = v > run_max
        return jnp.where(upd, v, run_max), jnp.where(upd, base + n, run_idx)

    def step(i, chains):
        return tuple(scan_vec(i * U + k, chains[k]) for k in range(U))

    init = tuple(
        (jnp.full((L,), NEG_INF, jnp.float32), jnp.zeros((L,), jnp.int32))
        for _ in range(U)
    )
    nfull = (NV - 1) // U
    chains = lax.fori_loop(0, nfull, step, init)

    chains = tuple(
        scan_vec(nfull * U + k, chains[k]) if nfull * U + k < NV - 1 else chains[k]
        for k in range(U)
    )

    run_max, run_idx = chains[0]
    for m2, i2 in chains[1:]:
        upd = (m2 > run_max) | ((m2 == run_max) & (i2 < run_idx))
        run_max = jnp.where(upd, m2, run_max)
        run_idx = jnp.where(upd, i2, run_idx)

    n = (NV - 1) * L + lane
    v = plsc.load_gather(sbuf, [jnp.minimum(n, C - 1), ones])
    v = jnp.where((v > 0.0) & (n < C), v, NEG_INF)
    upd = (v > run_max) | ((v == run_max) & (base + n < run_idx))
    run_max = jnp.where(upd, v, run_max)
    run_idx = jnp.where(upd, base + n, run_idx)

    tmpf[...] = run_max
    tmpi[...] = run_idx
    pltpu.sync_copy(tmpf, mstage.at[pl.ds(s * L, L)])
    pltpu.sync_copy(tmpi, istage.at[pl.ds(s * L, L)])
    plsc.subcore_barrier()

    @pl.when(w == 0)
    def _():
        pltpu.sync_copy(mstage.at[pl.ds(g * WPB * L, WPB * L)], mload)
        pltpu.sync_copy(istage.at[pl.ds(g * WPB * L, WPB * L)], iload)
        best_m = mload[pl.ds(0, L)]
        best_i = iload[pl.ds(0, L)]
        for k in range(1, WPB):
            m2 = mload[pl.ds(k * L, L)]
            i2 = iload[pl.ds(k * L, L)]
            upd = (m2 > best_m) | ((m2 == best_m) & (i2 < best_i))
            best_m = jnp.where(upd, m2, best_m)
            best_i = jnp.where(upd, i2, best_i)
        top = jnp.max(best_m, axis=0)
        cand = jnp.where(best_m == top, best_i, IMAX)
        idx = jnp.min(cand, axis=0)

        cl = pltpu.async_copy(landms_hbm.at[b, pl.ds(idx, 1), :], lrow, sem_l)
        cp = pltpu.async_copy(priors_hbm.at[pl.ds(idx, 1), :], prow, sem_p)
        cl.wait()
        cp.wait()

        zeros = jnp.zeros((L,), jnp.int32)
        par = lane & 1
        lvec = plsc.load_gather(lrow, [zeros, jnp.minimum(lane, 9)])
        pxy = plsc.load_gather(prow, [zeros, par])
        pwh = plsc.load_gather(prow, [zeros, par + 2])

        obuf[...] = pxy + lvec * VAR0 * pwh
        pltpu.sync_copy(obuf, out_hbm.at[b])


_sc_call = pl.kernel(
    _sc_body,
    out_type=jax.ShapeDtypeStruct((B, L), jnp.float32),
    mesh=_MESH,
    compiler_params=pltpu.CompilerParams(
        needs_layout_passes=False, use_tc_tiling_on_sc=False
    ),
    scratch_types=[
        pltpu.VMEM((NV * L, 2), jnp.float32),
        pltpu.VMEM_SHARED((NS * L,), jnp.float32),
        pltpu.VMEM_SHARED((NS * L,), jnp.int32),
        pltpu.VMEM((WPB * L,), jnp.float32),
        pltpu.VMEM((WPB * L,), jnp.int32),
        pltpu.VMEM((L,), jnp.float32),
        pltpu.VMEM((L,), jnp.int32),
        pltpu.VMEM((1, 10), jnp.float32),
        pltpu.VMEM((1, 4), jnp.float32),
        pltpu.VMEM((L,), jnp.float32),
        pltpu.SemaphoreType.DMA,
        pltpu.SemaphoreType.DMA,
    ],
)


def kernel(loc, conf, landms, priors):
    del loc
    out = _sc_call(conf, landms, priors)
    return out[:, :10]


# speedup vs baseline: 1.3358x; 1.3358x over previous
import jax
import jax.numpy as jnp
import numpy as np
from jax import lax
from jax.experimental import pallas as pl
from jax.experimental.pallas import tpu as pltpu
from jax.experimental.pallas import tpu_sc as plsc

B = 4
N = 16800
L = 16
NC = 2
NS = 16
WPB = 8
C = N // WPB
U = 4
NV = -(-C // L)
VAR0 = np.float32(0.1)
NEG_INF = np.float32(-np.inf)
IMAX = np.int32(2**31 - 1)

_MESH = plsc.VectorSubcoreMesh(
    core_axis_name="c", subcore_axis_name="s", num_cores=NC, num_subcores=NS
)


def _sc_body(conf_hbm, landms_hbm, priors_hbm, out_hbm,
             sbuf, mstage, istage, mload, iload, tmpf, tmpi, lrow, prow, obuf,
             sem_l, sem_p):
    c = lax.axis_index("c")
    s = lax.axis_index("s")
    g = s // WPB
    w = s % WPB
    b = 2 * c + g
    base = w * C

    pltpu.sync_copy(
        conf_hbm.at[pl.ds((b * N + base) * 2, C * 2)], sbuf.at[pl.ds(0, C * 2)]
    )

    lane = lax.iota(jnp.int32, L)

    def scan_vec(j, carry):
        run_max, run_idx = carry
        n = j * L + lane
        v = plsc.load_gather(sbuf, [n * 2 + 1])
        v = jnp.where(v > 0.0, v, NEG_INF)
        upd = v > run_max
        return jnp.where(upd, v, run_max), jnp.where(upd, base + n, run_idx)

    def step(i, chains):
        return tuple(scan_vec(i * U + k, chains[k]) for k in range(U))

    init = tuple(
        (jnp.full((L,), NEG_INF, jnp.float32), jnp.zeros((L,), jnp.int32))
        for _ in range(U)
    )
    nfull = (NV - 1) // U
    chains = lax.fori_loop(0, nfull, step, init)

    chains = tuple(
        scan_vec(nfull * U + k, chains[k]) if nfull * U + k < NV - 1 else chains[k]
        for k in range(U)
    )

    run_max, run_idx = chains[0]
    for m2, i2 in chains[1:]:
        upd = (m2 > run_max) | ((m2 == run_max) & (i2 < run_idx))
        run_max = jnp.where(upd, m2, run_max)
        run_idx = jnp.where(upd, i2, run_idx)

    n = (NV - 1) * L + lane
    v = plsc.load_gather(sbuf, [jnp.minimum(n, C - 1) * 2 + 1])
    v = jnp.where((v > 0.0) & (n < C), v, NEG_INF)
    upd = (v > run_max) | ((v == run_max) & (base + n < run_idx))
    run_max = jnp.where(upd, v, run_max)
    run_idx = jnp.where(upd, base + n, run_idx)

    tmpf[...] = run_max
    tmpi[...] = run_idx
    pltpu.sync_copy(tmpf, mstage.at[pl.ds(s * L, L)])
    pltpu.sync_copy(tmpi, istage.at[pl.ds(s * L, L)])
    plsc.subcore_barrier()

    @pl.when(w == 0)
    def _():
        pltpu.sync_copy(mstage.at[pl.ds(g * WPB * L, WPB * L)], mload)
        pltpu.sync_copy(istage.at[pl.ds(g * WPB * L, WPB * L)], iload)
        best_m = mload[pl.ds(0, L)]
        best_i = iload[pl.ds(0, L)]
        for k in range(1, WPB):
            m2 = mload[pl.ds(k * L, L)]
            i2 = iload[pl.ds(k * L, L)]
            upd = (m2 > best_m) | ((m2 == best_m) & (i2 < best_i))
            best_m = jnp.where(upd, m2, best_m)
            best_i = jnp.where(upd, i2, best_i)
        top = jnp.max(best_m, axis=0)
        cand = jnp.where(best_m == top, best_i, IMAX)
        idx = jnp.min(cand, axis=0)

        loff = (b * N + idx) * 10
        la = (loff // 8) * 8
        lr = loff - la
        poff = idx * 4
        pa = (poff // 8) * 8
        pr = poff - pa
        cl = pltpu.async_copy(landms_hbm.at[pl.ds(la, 24)], lrow, sem_l)
        cp = pltpu.async_copy(priors_hbm.at[pl.ds(pa, 16)], prow, sem_p)
        cl.wait()
        cp.wait()

        par = lane & 1
        lvec = plsc.load_gather(lrow, [lr + jnp.minimum(lane, 9)])
        pxy = plsc.load_gather(prow, [pr + par])
        pwh = plsc.load_gather(prow, [pr + par + 2])

        obuf[...] = pxy + lvec * VAR0 * pwh
        pltpu.sync_copy(obuf, out_hbm.at[b])


_sc_call = pl.kernel(
    _sc_body,
    out_type=jax.ShapeDtypeStruct((B, L), jnp.float32),
    mesh=_MESH,
    compiler_params=pltpu.CompilerParams(
        needs_layout_passes=False, use_tc_tiling_on_sc=False
    ),
    scratch_types=[
        pltpu.VMEM((NV * L * 2,), jnp.float32),
        pltpu.VMEM_SHARED((NS * L,), jnp.float32),
        pltpu.VMEM_SHARED((NS * L,), jnp.int32),
        pltpu.VMEM((WPB * L,), jnp.float32),
        pltpu.VMEM((WPB * L,), jnp.int32),
        pltpu.VMEM((L,), jnp.float32),
        pltpu.VMEM((L,), jnp.int32),
        pltpu.VMEM((24,), jnp.float32),
        pltpu.VMEM((16,), jnp.float32),
        pltpu.VMEM((L,), jnp.float32),
        pltpu.SemaphoreType.DMA,
        pltpu.SemaphoreType.DMA,
    ],
)


def kernel(loc, conf, landms, priors):
    del loc
    out = _sc_call(conf.reshape(-1), landms.reshape(-1), priors.reshape(-1))
    return out[:, :10]


# speedup vs baseline: 5.2302x; 3.9154x over previous
import jax
import jax.numpy as jnp
import numpy as np
from jax import lax
from jax.experimental import pallas as pl
from jax.experimental.pallas import tpu as pltpu
from jax.experimental.pallas import tpu_sc as plsc

B = 4
N = 16800
L = 16
NC = 2
NS = 16
WPB = 8
C = N // WPB
U = 4
NV = -(-C // L)
VAR0 = np.float32(0.1)
NEG_INF = np.float32(-np.inf)
IMAX = np.int32(2**31 - 1)

_MESH = plsc.VectorSubcoreMesh(
    core_axis_name="c", subcore_axis_name="s", num_cores=NC, num_subcores=NS
)


def _sc_body(scores_hbm, landms_hbm, priors_hbm, out_hbm,
             sbuf, mstage, istage, mload, iload, tmpf, tmpi,
             ivl, ivx, ivw, lrow, pxyb, pwhb, obuf, sem_l, sem_x, sem_w):
    c = lax.axis_index("c")
    s = lax.axis_index("s")
    g = s // WPB
    w = s % WPB
    b = 2 * c + g
    base = w * C

    start = b * N + base
    a0 = (start // 8) * 8
    rem = start - a0
    pltpu.sync_copy(scores_hbm.at[pl.ds(a0, C + 4)], sbuf.at[pl.ds(0, C + 4)])

    lane = lax.iota(jnp.int32, L)
    lane_r = lane + rem

    def scan_vec(j, carry):
        run_max, run_idx = carry
        n = j * L + lane
        v = plsc.load_gather(sbuf, [j * L + lane_r])
        v = jnp.where(v > 0.0, v, NEG_INF)
        upd = v > run_max
        return jnp.where(upd, v, run_max), jnp.where(upd, base + n, run_idx)

    def step(i, chains):
        return tuple(scan_vec(i * U + k, chains[k]) for k in range(U))

    init = tuple(
        (jnp.full((L,), NEG_INF, jnp.float32), jnp.zeros((L,), jnp.int32))
        for _ in range(U)
    )
    nfull = (NV - 1) // U
    chains = lax.fori_loop(0, nfull, step, init)

    chains = tuple(
        scan_vec(nfull * U + k, chains[k]) if nfull * U + k < NV - 1 else chains[k]
        for k in range(U)
    )

    run_max, run_idx = chains[0]
    for m2, i2 in chains[1:]:
        upd = (m2 > run_max) | ((m2 == run_max) & (i2 < run_idx))
        run_max = jnp.where(upd, m2, run_max)
        run_idx = jnp.where(upd, i2, run_idx)

    n = (NV - 1) * L + lane
    v = plsc.load_gather(sbuf, [jnp.minimum(n, C - 1) + rem])
    v = jnp.where((v > 0.0) & (n < C), v, NEG_INF)
    upd = (v > run_max) | ((v == run_max) & (base + n < run_idx))
    run_max = jnp.where(upd, v, run_max)
    run_idx = jnp.where(upd, base + n, run_idx)

    tmpf[...] = run_max
    tmpi[...] = run_idx
    pltpu.sync_copy(tmpf, mstage.at[pl.ds(s * L, L)])
    pltpu.sync_copy(tmpi, istage.at[pl.ds(s * L, L)])
    plsc.subcore_barrier()

    @pl.when(w == 0)
    def _():
        pltpu.sync_copy(mstage.at[pl.ds(g * WPB * L, WPB * L)], mload)
        pltpu.sync_copy(istage.at[pl.ds(g * WPB * L, WPB * L)], iload)
        best_m = mload[pl.ds(0, L)]
        best_i = iload[pl.ds(0, L)]
        for k in range(1, WPB):
            m2 = mload[pl.ds(k * L, L)]
            i2 = iload[pl.ds(k * L, L)]
            upd = (m2 > best_m) | ((m2 == best_m) & (i2 < best_i))
            best_m = jnp.where(upd, m2, best_m)
            best_i = jnp.where(upd, i2, best_i)
        top = jnp.max(best_m, axis=0)
        cand = jnp.where(best_m == top, best_i, IMAX)
        idx = jnp.min(cand, axis=0)

        par = lane & 1
        ivl[...] = (jnp.minimum(lane, 9) * B + b) * N + idx
        ivx[...] = par * N + idx
        ivw[...] = (par + 2) * N + idx
        cl = pltpu.async_copy(landms_hbm.at[ivl], lrow, sem_l)
        cx = pltpu.async_copy(priors_hbm.at[ivx], pxyb, sem_x)
        cw = pltpu.async_copy(priors_hbm.at[ivw], pwhb, sem_w)
        cl.wait()
        cx.wait()
        cw.wait()

        obuf[...] = pxyb[...] + lrow[...] * VAR0 * pwhb[...]
        pltpu.sync_copy(obuf, out_hbm.at[b])


_sc_call = pl.kernel(
    _sc_body,
    out_type=jax.ShapeDtypeStruct((B, L), jnp.float32),
    mesh=_MESH,
    compiler_params=pltpu.CompilerParams(
        needs_layout_passes=False, use_tc_tiling_on_sc=False
    ),
    scratch_types=[
        pltpu.VMEM((NV * L + 8,), jnp.float32),
        pltpu.VMEM_SHARED((NS * L,), jnp.float32),
        pltpu.VMEM_SHARED((NS * L,), jnp.int32),
        pltpu.VMEM((WPB * L,), jnp.float32),
        pltpu.VMEM((WPB * L,), jnp.int32),
        pltpu.VMEM((L,), jnp.float32),
        pltpu.VMEM((L,), jnp.int32),
        pltpu.VMEM((L,), jnp.int32),
        pltpu.VMEM((L,), jnp.int32),
        pltpu.VMEM((L,), jnp.int32),
        pltpu.VMEM((L,), jnp.float32),
        pltpu.VMEM((L,), jnp.float32),
        pltpu.VMEM((L,), jnp.float32),
        pltpu.VMEM((L,), jnp.float32),
        pltpu.SemaphoreType.DMA,
        pltpu.SemaphoreType.DMA,
        pltpu.SemaphoreType.DMA,
    ],
)


def kernel(loc, conf, landms, priors):
    del loc
    scores = conf[:, :, 1].reshape(-1)
    landms_f = landms.transpose(2, 0, 1).reshape(-1)
    priors_f = priors.transpose(1, 0).reshape(-1)
    out = _sc_call(scores, landms_f, priors_f)
    return out[:, :10]


# speedup vs baseline: 5.2456x; 1.0029x over previous
import jax
import jax.numpy as jnp
import numpy as np
from jax import lax
from jax.experimental import pallas as pl
from jax.experimental.pallas import tpu as pltpu
from jax.experimental.pallas import tpu_sc as plsc

B = 4
N = 16800
L = 16
NC = 2
NS = 16
WPB = 8
C = N // WPB
U = 4
NV = -(-C // L)
VAR0 = np.float32(0.1)
NEG_INF = np.float32(-np.inf)
IMAX = np.int32(2**31 - 1)

_MESH = plsc.VectorSubcoreMesh(
    core_axis_name="c", subcore_axis_name="s", num_cores=NC, num_subcores=NS
)


def _sc_body(scores_hbm, landms_hbm, priors_hbm, out_hbm,
             sbuf, mstage, istage, mload, iload, tmpf, tmpi,
             ivl, ivx, ivw, lrow, pxyb, pwhb, obuf, sem_l, sem_x, sem_w):
    c = lax.axis_index("c")
    s = lax.axis_index("s")
    g = s // WPB
    w = s % WPB
    b = 2 * c + g
    base = w * C

    start = b * N + base
    a0 = (start // 8) * 8
    rem = start - a0
    pltpu.sync_copy(scores_hbm.at[pl.ds(a0, C + 4)], sbuf.at[pl.ds(0, C + 4)])

    lane = lax.iota(jnp.int32, L)
    lane_r = lane + rem

    def scan_vec(j, carry):
        run_max, run_idx = carry
        n = j * L + lane
        v = plsc.load_gather(sbuf, [j * L + lane_r])
        v = jnp.where(v > 0.0, v, NEG_INF)
        upd = v > run_max
        return jnp.where(upd, v, run_max), jnp.where(upd, base + n, run_idx)

    def step(i, chains):
        return tuple(scan_vec(i * U + k, chains[k]) for k in range(U))

    init = tuple(
        (jnp.full((L,), NEG_INF, jnp.float32), jnp.zeros((L,), jnp.int32))
        for _ in range(U)
    )
    nfull = (NV - 1) // U
    chains = lax.fori_loop(0, nfull, step, init)

    chains = tuple(
        scan_vec(nfull * U + k, chains[k]) if nfull * U + k < NV - 1 else chains[k]
        for k in range(U)
    )

    run_max, run_idx = chains[0]
    for m2, i2 in chains[1:]:
        upd = (m2 > run_max) | ((m2 == run_max) & (i2 < run_idx))
        run_max = jnp.where(upd, m2, run_max)
        run_idx = jnp.where(upd, i2, run_idx)

    n = (NV - 1) * L + lane
    v = plsc.load_gather(sbuf, [jnp.minimum(n, C - 1) + rem])
    v = jnp.where((v > 0.0) & (n < C), v, NEG_INF)
    upd = (v > run_max) | ((v == run_max) & (base + n < run_idx))
    run_max = jnp.where(upd, v, run_max)
    run_idx = jnp.where(upd, base + n, run_idx)

    tmpf[...] = run_max
    tmpi[...] = run_idx
    pltpu.sync_copy(tmpf, mstage.at[pl.ds(s * L, L)])
    pltpu.sync_copy(tmpi, istage.at[pl.ds(s * L, L)])
    plsc.subcore_barrier()

    @pl.when(w == 0)
    def _():
        pltpu.sync_copy(mstage.at[pl.ds(g * WPB * L, WPB * L)], mload)
        pltpu.sync_copy(istage.at[pl.ds(g * WPB * L, WPB * L)], iload)
        best_m = mload[pl.ds(0, L)]
        best_i = iload[pl.ds(0, L)]
        for k in range(1, WPB):
            m2 = mload[pl.ds(k * L, L)]
            i2 = iload[pl.ds(k * L, L)]
            upd = (m2 > best_m) | ((m2 == best_m) & (i2 < best_i))
            best_m = jnp.where(upd, m2, best_m)
            best_i = jnp.where(upd, i2, best_i)
        top = jnp.max(best_m, axis=0)
        cand = jnp.where(best_m == top, best_i, IMAX)
        idx = jnp.min(cand, axis=0)

        par = lane & 1
        ivl[...] = (jnp.minimum(lane, 9) * B + b) * N + idx
        ivx[...] = par * N + idx
        ivw[...] = (par + 2) * N + idx
        cl = pltpu.async_copy(landms_hbm.at[ivl], lrow, sem_l)
        cx = pltpu.async_copy(priors_hbm.at[ivx], pxyb, sem_x)
        cw = pltpu.async_copy(priors_hbm.at[ivw], pwhb, sem_w)
        cl.wait()
        cx.wait()
        cw.wait()

        obuf[...] = pxyb[...] + lrow[...] * VAR0 * pwhb[...]
        pltpu.sync_copy(obuf, out_hbm.at[b])


_sc_call = pl.kernel(
    _sc_body,
    out_type=jax.ShapeDtypeStruct((B, L), jnp.float32),
    mesh=_MESH,
    compiler_params=pltpu.CompilerParams(
        needs_layout_passes=False,
        use_tc_tiling_on_sc=False,
        skip_device_barrier=True,
    ),
    scratch_types=[
        pltpu.VMEM((NV * L + 8,), jnp.float32),
        pltpu.VMEM_SHARED((NS * L,), jnp.float32),
        pltpu.VMEM_SHARED((NS * L,), jnp.int32),
        pltpu.VMEM((WPB * L,), jnp.float32),
        pltpu.VMEM((WPB * L,), jnp.int32),
        pltpu.VMEM((L,), jnp.float32),
        pltpu.VMEM((L,), jnp.int32),
        pltpu.VMEM((L,), jnp.int32),
        pltpu.VMEM((L,), jnp.int32),
        pltpu.VMEM((L,), jnp.int32),
        pltpu.VMEM((L,), jnp.float32),
        pltpu.VMEM((L,), jnp.float32),
        pltpu.VMEM((L,), jnp.float32),
        pltpu.VMEM((L,), jnp.float32),
        pltpu.SemaphoreType.DMA,
        pltpu.SemaphoreType.DMA,
        pltpu.SemaphoreType.DMA,
    ],
)


def kernel(loc, conf, landms, priors):
    del loc
    scores = conf[:, :, 1].reshape(-1)
    landms_f = landms.transpose(2, 0, 1).reshape(-1)
    priors_f = priors.transpose(1, 0).reshape(-1)
    out = _sc_call(scores, landms_f, priors_f)
    return out[:, :10]


# speedup vs baseline: 6.2722x; 1.1957x over previous
import jax
import jax.numpy as jnp
import numpy as np
from jax import lax
from jax.experimental import pallas as pl
from jax.experimental.pallas import tpu as pltpu
from jax.experimental.pallas import tpu_sc as plsc

B = 4
N = 16800
L = 16
NC = 2
NS = 16
WPB = 8
C = N // WPB
U = 4
NV = -(-C // L)
VAR0 = np.float32(0.1)
NEG_INF = np.float32(-np.inf)
IMAX = np.int32(2**31 - 1)

_MESH = plsc.VectorSubcoreMesh(
    core_axis_name="c", subcore_axis_name="s", num_cores=NC, num_subcores=NS
)


def _sc_body(scores_hbm, out_hbm, sbuf, mstage, istage, mload, iload, tmpf, tmpi):
    c = lax.axis_index("c")
    s = lax.axis_index("s")
    g = s // WPB
    w = s % WPB
    b = 2 * c + g
    base = w * C

    start = b * N + base
    a0 = (start // 8) * 8
    rem = start - a0
    pltpu.sync_copy(scores_hbm.at[pl.ds(a0, C + 4)], sbuf.at[pl.ds(0, C + 4)])

    lane = lax.iota(jnp.int32, L)
    lane_r = lane + rem

    def scan_vec(j, carry):
        run_max, run_idx = carry
        n = j * L + lane
        v = plsc.load_gather(sbuf, [j * L + lane_r])
        v = jnp.where(v > 0.0, v, NEG_INF)
        upd = v > run_max
        return jnp.where(upd, v, run_max), jnp.where(upd, base + n, run_idx)

    def step(i, chains):
        return tuple(scan_vec(i * U + k, chains[k]) for k in range(U))

    init = tuple(
        (jnp.full((L,), NEG_INF, jnp.float32), jnp.zeros((L,), jnp.int32))
        for _ in range(U)
    )
    nfull = (NV - 1) // U
    chains = lax.fori_loop(0, nfull, step, init)

    chains = tuple(
        scan_vec(nfull * U + k, chains[k]) if nfull * U + k < NV - 1 else chains[k]
        for k in range(U)
    )

    run_max, run_idx = chains[0]
    for m2, i2 in chains[1:]:
        upd = (m2 > run_max) | ((m2 == run_max) & (i2 < run_idx))
        run_max = jnp.where(upd, m2, run_max)
        run_idx = jnp.where(upd, i2, run_idx)

    n = (NV - 1) * L + lane
    v = plsc.load_gather(sbuf, [jnp.minimum(n, C - 1) + rem])
    v = jnp.where((v > 0.0) & (n < C), v, NEG_INF)
    upd = (v > run_max) | ((v == run_max) & (base + n < run_idx))
    run_max = jnp.where(upd, v, run_max)
    run_idx = jnp.where(upd, base + n, run_idx)

    tmpf[...] = run_max
    tmpi[...] = run_idx
    pltpu.sync_copy(tmpf, mstage.at[pl.ds(s * L, L)])
    pltpu.sync_copy(tmpi, istage.at[pl.ds(s * L, L)])
    plsc.subcore_barrier()

    @pl.when(w == 0)
    def _():
        pltpu.sync_copy(mstage.at[pl.ds(g * WPB * L, WPB * L)], mload)
        pltpu.sync_copy(istage.at[pl.ds(g * WPB * L, WPB * L)], iload)
        best_m = mload[pl.ds(0, L)]
        best_i = iload[pl.ds(0, L)]
        for k in range(1, WPB):
            m2 = mload[pl.ds(k * L, L)]
            i2 = iload[pl.ds(k * L, L)]
            upd = (m2 > best_m) | ((m2 == best_m) & (i2 < best_i))
            best_m = jnp.where(upd, m2, best_m)
            best_i = jnp.where(upd, i2, best_i)
        top = jnp.max(best_m, axis=0)
        cand = jnp.where(best_m == top, best_i, IMAX)
        tmpi[...] = jnp.min(cand, keepdims=True) + jnp.zeros((L,), jnp.int32)
        pltpu.sync_copy(tmpi, out_hbm.at[b])


_sc_call = pl.kernel(
    _sc_body,
    out_type=jax.ShapeDtypeStruct((B, L), jnp.int32),
    mesh=_MESH,
    compiler_params=pltpu.CompilerParams(
        needs_layout_passes=False, use_tc_tiling_on_sc=False
    ),
    scratch_types=[
        pltpu.VMEM((NV * L + 8,), jnp.float32),
        pltpu.VMEM_SHARED((NS * L,), jnp.float32),
        pltpu.VMEM_SHARED((NS * L,), jnp.int32),
        pltpu.VMEM((WPB * L,), jnp.float32),
        pltpu.VMEM((WPB * L,), jnp.int32),
        pltpu.VMEM((L,), jnp.float32),
        pltpu.VMEM((L,), jnp.int32),
    ],
)


def _tc_body(idx_ref, landms_ref, priors_ref, out_ref):
    idxv = idx_ref[...][:, 0:1]
    nio = lax.broadcasted_iota(jnp.int32, (B, N), 1)
    mask = (nio == idxv).astype(jnp.float32)
    kpar = lax.broadcasted_iota(jnp.int32, (1, L), 1) & 1
    contract = (((1,), (1,)), ((), ()))
    priors_all = priors_ref[...]
    for b in range(B):
        m_b = mask[b : b + 1, :]
        lv = lax.dot_general(
            m_b, landms_ref[:, b, :], contract,
            preferred_element_type=jnp.float32,
        )
        pr = lax.dot_general(
            m_b, priors_all, contract,
            preferred_element_type=jnp.float32,
        )
        lv16 = jnp.concatenate([lv, jnp.zeros((1, L - 10), jnp.float32)], axis=1)
        pxy = jnp.where(kpar == 0, pr[:, 0:1], pr[:, 1:2])
        pwh = jnp.where(kpar == 0, pr[:, 2:3], pr[:, 3:4])
        out_ref[b : b + 1, :] = pxy + lv16 * VAR0 * pwh


_tc_call = pl.pallas_call(
    _tc_body,
    out_shape=jax.ShapeDtypeStruct((B, L), jnp.float32),
)


def kernel(loc, conf, landms, priors):
    del loc
    scores = conf[:, :, 1].reshape(-1)
    idx_arr = _sc_call(scores)
    landms_t = landms.transpose(2, 0, 1)
    priors_t = priors.transpose(1, 0)
    out = _tc_call(idx_arr, landms_t, priors_t)
    return out[:, :10]


# speedup vs baseline: 6.6041x; 1.0529x over previous
import jax
import jax.numpy as jnp
import numpy as np
from jax import lax
from jax.experimental import pallas as pl
from jax.experimental.pallas import tpu as pltpu
from jax.experimental.pallas import tpu_sc as plsc

B = 4
N = 16800
L = 16
NC = 1
NS = 16
WPB = 4
C = N // WPB
U = 4
NV = -(-C // L)
VAR0 = np.float32(0.1)
NEG_INF = np.float32(-np.inf)
IMAX = np.int32(2**31 - 1)

_MESH = plsc.VectorSubcoreMesh(
    core_axis_name="c", subcore_axis_name="s", num_cores=NC, num_subcores=NS
)


def _sc_body(scores_hbm, out_hbm, sbuf, mstage, istage, mload, iload, tmpf, tmpi):
    s = lax.axis_index("s")
    g = s // WPB
    w = s % WPB
    b = g
    base = w * C

    start = b * N + base
    a0 = (start // 8) * 8
    rem = start - a0
    pltpu.sync_copy(scores_hbm.at[pl.ds(a0, C + 4)], sbuf.at[pl.ds(0, C + 4)])

    lane = lax.iota(jnp.int32, L)
    lane_r = lane + rem

    def scan_vec(j, carry):
        run_max, run_idx = carry
        n = j * L + lane
        v = plsc.load_gather(sbuf, [j * L + lane_r])
        v = jnp.where(v > 0.0, v, NEG_INF)
        upd = v > run_max
        return jnp.where(upd, v, run_max), jnp.where(upd, base + n, run_idx)

    def step(i, chains):
        return tuple(scan_vec(i * U + k, chains[k]) for k in range(U))

    init = tuple(
        (jnp.full((L,), NEG_INF, jnp.float32), jnp.zeros((L,), jnp.int32))
        for _ in range(U)
    )
    nfull = (NV - 1) // U
    chains = lax.fori_loop(0, nfull, step, init)

    chains = tuple(
        scan_vec(nfull * U + k, chains[k]) if nfull * U + k < NV - 1 else chains[k]
        for k in range(U)
    )

    run_max, run_idx = chains[0]
    for m2, i2 in chains[1:]:
        upd = (m2 > run_max) | ((m2 == run_max) & (i2 < run_idx))
        run_max = jnp.where(upd, m2, run_max)
        run_idx = jnp.where(upd, i2, run_idx)

    n = (NV - 1) * L + lane
    v = plsc.load_gather(sbuf, [jnp.minimum(n, C - 1) + rem])
    v = jnp.where((v > 0.0) & (n < C), v, NEG_INF)
    upd = (v > run_max) | ((v == run_max) & (base + n < run_idx))
    run_max = jnp.where(upd, v, run_max)
    run_idx = jnp.where(upd, base + n, run_idx)

    tmpf[...] = run_max
    tmpi[...] = run_idx
    pltpu.sync_copy(tmpf, mstage.at[pl.ds(s * L, L)])
    pltpu.sync_copy(tmpi, istage.at[pl.ds(s * L, L)])
    plsc.subcore_barrier()

    @pl.when(w == 0)
    def _():
        pltpu.sync_copy(mstage.at[pl.ds(g * WPB * L, WPB * L)], mload)
        pltpu.sync_copy(istage.at[pl.ds(g * WPB * L, WPB * L)], iload)
        best_m = mload[pl.ds(0, L)]
        best_i = iload[pl.ds(0, L)]
        for k in range(1, WPB):
            m2 = mload[pl.ds(k * L, L)]
            i2 = iload[pl.ds(k * L, L)]
            upd = (m2 > best_m) | ((m2 == best_m) & (i2 < best_i))
            best_m = jnp.where(upd, m2, best_m)
            best_i = jnp.where(upd, i2, best_i)
        top = jnp.max(best_m, axis=0)
        cand = jnp.where(best_m == top, best_i, IMAX)
        tmpi[...] = jnp.min(cand, keepdims=True) + jnp.zeros((L,), jnp.int32)
        pltpu.sync_copy(tmpi, out_hbm.at[b])


_sc_call = pl.kernel(
    _sc_body,
    out_type=jax.ShapeDtypeStruct((B, L), jnp.int32),
    mesh=_MESH,
    compiler_params=pltpu.CompilerParams(
        needs_layout_passes=False, use_tc_tiling_on_sc=False
    ),
    scratch_types=[
        pltpu.VMEM((NV * L + 8,), jnp.float32),
        pltpu.VMEM_SHARED((NS * L,), jnp.float32),
        pltpu.VMEM_SHARED((NS * L,), jnp.int32),
        pltpu.VMEM((WPB * L,), jnp.float32),
        pltpu.VMEM((WPB * L,), jnp.int32),
        pltpu.VMEM((L,), jnp.float32),
        pltpu.VMEM((L,), jnp.int32),
    ],
)


def _tc_body(idx_ref, landms_ref, priors_ref, out_ref):
    idxv = idx_ref[...][:, 0:1]
    nio = lax.broadcasted_iota(jnp.int32, (B, N), 1)
    mask = (nio == idxv).astype(jnp.float32)
    kpar = lax.broadcasted_iota(jnp.int32, (1, L), 1) & 1
    contract = (((1,), (1,)), ((), ()))
    priors_all = priors_ref[...]
    for b in range(B):
        m_b = mask[b : b + 1, :]
        lv = lax.dot_general(
            m_b, landms_ref[:, b, :], contract,
            preferred_element_type=jnp.float32,
        )
        pr = lax.dot_general(
            m_b, priors_all, contract,
            preferred_element_type=jnp.float32,
        )
        lv16 = jnp.concatenate([lv, jnp.zeros((1, L - 10), jnp.float32)], axis=1)
        pxy = jnp.where(kpar == 0, pr[:, 0:1], pr[:, 1:2])
        pwh = jnp.where(kpar == 0, pr[:, 2:3], pr[:, 3:4])
        out_ref[b : b + 1, :] = pxy + lv16 * VAR0 * pwh


_tc_call = pl.pallas_call(
    _tc_body,
    out_shape=jax.ShapeDtypeStruct((B, L), jnp.float32),
)


def kernel(loc, conf, landms, priors):
    del loc
    scores = conf[:, :, 1].reshape(-1)
    idx_arr = _sc_call(scores)
    landms_t = landms.transpose(2, 0, 1)
    priors_t = priors.transpose(1, 0)
    out = _tc_call(idx_arr, landms_t, priors_t)
    return out[:, :10]


# speedup vs baseline: 6.6712x; 1.0102x over previous
import jax
import jax.numpy as jnp
import numpy as np
from jax import lax
from jax.experimental import pallas as pl
from jax.experimental.pallas import tpu as pltpu
from jax.experimental.pallas import tpu_sc as plsc

B = 4
N = 16800
L = 16
NC = 1
NS = 16
WPB = 4
C = N // WPB
U = 4
NV = -(-C // L)
VAR0 = np.float32(0.1)
NEG_INF = np.float32(-np.inf)
IMAX = np.int32(2**31 - 1)

_MESH = plsc.VectorSubcoreMesh(
    core_axis_name="c", subcore_axis_name="s", num_cores=NC, num_subcores=NS
)


def _sc_body(scores_hbm, out_hbm, sbuf, mstage, istage, mload, iload, tmpf, tmpi):
    s = lax.axis_index("s")
    g = s // WPB
    w = s % WPB
    b = g
    base = w * C

    start = b * N + base
    a0 = (start // 8) * 8
    rem = start - a0
    pltpu.sync_copy(scores_hbm.at[pl.ds(a0, C + 4)], sbuf.at[pl.ds(0, C + 4)])

    lane = lax.iota(jnp.int32, L)
    lane_r = lane + rem

    def scan_vec(j, carry):
        run_max, run_idx = carry
        n = j * L + lane
        v = plsc.load_gather(sbuf, [j * L + lane_r])
        v = jnp.where(v > 0.0, v, NEG_INF)
        upd = v > run_max
        return jnp.where(upd, v, run_max), jnp.where(upd, base + n, run_idx)

    def step(i, chains):
        return tuple(scan_vec(i * U + k, chains[k]) for k in range(U))

    init = tuple(
        (jnp.full((L,), NEG_INF, jnp.float32), jnp.zeros((L,), jnp.int32))
        for _ in range(U)
    )
    nfull = (NV - 1) // U
    chains = lax.fori_loop(0, nfull, step, init)

    chains = tuple(
        scan_vec(nfull * U + k, chains[k]) if nfull * U + k < NV - 1 else chains[k]
        for k in range(U)
    )

    run_max, run_idx = chains[0]
    for m2, i2 in chains[1:]:
        upd = (m2 > run_max) | ((m2 == run_max) & (i2 < run_idx))
        run_max = jnp.where(upd, m2, run_max)
        run_idx = jnp.where(upd, i2, run_idx)

    n = (NV - 1) * L + lane
    v = plsc.load_gather(sbuf, [jnp.minimum(n, C - 1) + rem])
    v = jnp.where((v > 0.0) & (n < C), v, NEG_INF)
    upd = (v > run_max) | ((v == run_max) & (base + n < run_idx))
    run_max = jnp.where(upd, v, run_max)
    run_idx = jnp.where(upd, base + n, run_idx)

    tmpf[...] = run_max
    tmpi[...] = run_idx
    pltpu.sync_copy(tmpf, mstage.at[pl.ds(s * L, L)])
    pltpu.sync_copy(tmpi, istage.at[pl.ds(s * L, L)])
    plsc.subcore_barrier()

    @pl.when(w == 0)
    def _():
        pltpu.sync_copy(mstage.at[pl.ds(g * WPB * L, WPB * L)], mload)
        pltpu.sync_copy(istage.at[pl.ds(g * WPB * L, WPB * L)], iload)
        best_m = mload[pl.ds(0, L)]
        best_i = iload[pl.ds(0, L)]
        for k in range(1, WPB):
            m2 = mload[pl.ds(k * L, L)]
            i2 = iload[pl.ds(k * L, L)]
            upd = (m2 > best_m) | ((m2 == best_m) & (i2 < best_i))
            best_m = jnp.where(upd, m2, best_m)
            best_i = jnp.where(upd, i2, best_i)
        top = jnp.max(best_m, axis=0)
        cand = jnp.where(best_m == top, best_i, IMAX)
        tmpi[...] = jnp.min(cand, keepdims=True) + jnp.zeros((L,), jnp.int32)
        pltpu.sync_copy(tmpi, out_hbm.at[b])


_sc_call = pl.kernel(
    _sc_body,
    out_type=jax.ShapeDtypeStruct((B, L), jnp.int32),
    mesh=_MESH,
    compiler_params=pltpu.CompilerParams(
        needs_layout_passes=False, use_tc_tiling_on_sc=False
    ),
    scratch_types=[
        pltpu.VMEM((NV * L + 8,), jnp.float32),
        pltpu.VMEM_SHARED((NS * L,), jnp.float32),
        pltpu.VMEM_SHARED((NS * L,), jnp.int32),
        pltpu.VMEM((WPB * L,), jnp.float32),
        pltpu.VMEM((WPB * L,), jnp.int32),
        pltpu.VMEM((L,), jnp.float32),
        pltpu.VMEM((L,), jnp.int32),
    ],
)


_BLK = 128


def _tc_body(idx_ref, landms_ref, priors_ref, out_ref):
    b = pl.program_id(0)
    rel = lax.rem(idx_ref[b, 0], _BLK)
    contract = (((1,), (1,)), ((), ()))
    mask1 = (
        lax.broadcasted_iota(jnp.int32, (1, _BLK), 1) == rel
    ).astype(jnp.float32)
    bmask = (
        lax.broadcasted_iota(jnp.int32, (B, 1), 0) == b
    ).astype(jnp.float32)
    lv2 = jnp.sum(landms_ref[...] * bmask[None, :, :], axis=1)
    lv = lax.dot_general(
        mask1, lv2, contract, precision=lax.Precision.HIGHEST,
        preferred_element_type=jnp.float32,
    )
    pr = lax.dot_general(
        mask1, priors_ref[...], contract, precision=lax.Precision.HIGHEST,
        preferred_element_type=jnp.float32,
    )
    lv16 = jnp.concatenate([lv, jnp.zeros((1, L - 10), jnp.float32)], axis=1)
    kpar = lax.broadcasted_iota(jnp.int32, (1, L), 1) & 1
    pxy = jnp.where(kpar == 0, pr[:, 0:1], pr[:, 1:2])
    pwh = jnp.where(kpar == 0, pr[:, 2:3], pr[:, 3:4])
    out_ref[...] = (pxy + lv16 * VAR0 * pwh).reshape(1, 1, L)


_tc_call = pl.pallas_call(
    _tc_body,
    grid_spec=pltpu.PrefetchScalarGridSpec(
        num_scalar_prefetch=1,
        grid=(B,),
        in_specs=[
            pl.BlockSpec(
                (10, B, _BLK), lambda b, idx_ref: (0, 0, idx_ref[b, 0] // _BLK)
            ),
            pl.BlockSpec((B, _BLK), lambda b, idx_ref: (0, idx_ref[b, 0] // _BLK)),
        ],
        out_specs=pl.BlockSpec((1, 1, L), lambda b, idx_ref: (b, 0, 0)),
    ),
    out_shape=jax.ShapeDtypeStruct((B, 1, L), jnp.float32),
)


def kernel(loc, conf, landms, priors):
    del loc
    scores = conf[:, :, 1].reshape(-1)
    idx_arr = _sc_call(scores)
    landms_t = landms.transpose(2, 0, 1)
    priors_t = priors.transpose(1, 0)
    out = _tc_call(idx_arr, landms_t, priors_t)
    return out[:, 0, :10]


# speedup vs baseline: 6.7154x; 1.0066x over previous
import jax
import jax.numpy as jnp
import numpy as np
from jax import lax
from jax.experimental import pallas as pl
from jax.experimental.pallas import tpu as pltpu
from jax.experimental.pallas import tpu_sc as plsc

B = 4
N = 16800
L = 16
NC = 1
NS = 16
WPB = 4
C = N // WPB
U = 4
NV = -(-C // L)
VAR0 = np.float32(0.1)
NEG_INF = np.float32(-np.inf)
IMAX = np.int32(2**31 - 1)

_MESH = plsc.VectorSubcoreMesh(
    core_axis_name="c", subcore_axis_name="s", num_cores=NC, num_subcores=NS
)


def _sc_body(scores_hbm, out_hbm, sbuf, mstage, istage, mload, iload, tmpf, tmpi):
    s = lax.axis_index("s")
    g = s // WPB
    w = s % WPB
    b = g
    base = w * C

    start = (2 * b + 1) * N + base
    a0 = (start // 8) * 8
    rem = start - a0
    pltpu.sync_copy(scores_hbm.at[pl.ds(a0, C + 4)], sbuf.at[pl.ds(0, C + 4)])

    lane = lax.iota(jnp.int32, L)
    lane_r = lane + rem

    def scan_vec(j, carry):
        run_max, run_idx = carry
        n = j * L + lane
        v = plsc.load_gather(sbuf, [j * L + lane_r])
        v = jnp.where(v > 0.0, v, NEG_INF)
        upd = v > run_max
        return jnp.where(upd, v, run_max), jnp.where(upd, base + n, run_idx)

    def step(i, chains):
        return tuple(scan_vec(i * U + k, chains[k]) for k in range(U))

    init = tuple(
        (jnp.full((L,), NEG_INF, jnp.float32), jnp.zeros((L,), jnp.int32))
        for _ in range(U)
    )
    nfull = (NV - 1) // U
    chains = lax.fori_loop(0, nfull, step, init)

    chains = tuple(
        scan_vec(nfull * U + k, chains[k]) if nfull * U + k < NV - 1 else chains[k]
        for k in range(U)
    )

    run_max, run_idx = chains[0]
    for m2, i2 in chains[1:]:
        upd = (m2 > run_max) | ((m2 == run_max) & (i2 < run_idx))
        run_max = jnp.where(upd, m2, run_max)
        run_idx = jnp.where(upd, i2, run_idx)

    n = (NV - 1) * L + lane
    v = plsc.load_gather(sbuf, [jnp.minimum(n, C - 1) + rem])
    v = jnp.where((v > 0.0) & (n < C), v, NEG_INF)
    upd = (v > run_max) | ((v == run_max) & (base + n < run_idx))
    run_max = jnp.where(upd, v, run_max)
    run_idx = jnp.where(upd, base + n, run_idx)

    tmpf[...] = run_max
    tmpi[...] = run_idx
    pltpu.sync_copy(tmpf, mstage.at[pl.ds(s * L, L)])
    pltpu.sync_copy(tmpi, istage.at[pl.ds(s * L, L)])
    plsc.subcore_barrier()

    @pl.when(w == 0)
    def _():
        pltpu.sync_copy(mstage.at[pl.ds(g * WPB * L, WPB * L)], mload)
        pltpu.sync_copy(istage.at[pl.ds(g * WPB * L, WPB * L)], iload)
        best_m = mload[pl.ds(0, L)]
        best_i = iload[pl.ds(0, L)]
        for k in range(1, WPB):
            m2 = mload[pl.ds(k * L, L)]
            i2 = iload[pl.ds(k * L, L)]
            upd = (m2 > best_m) | ((m2 == best_m) & (i2 < best_i))
            best_m = jnp.where(upd, m2, best_m)
            best_i = jnp.where(upd, i2, best_i)
        top = jnp.max(best_m, axis=0)
        cand = jnp.where(best_m == top, best_i, IMAX)
        tmpi[...] = jnp.min(cand, keepdims=True) + jnp.zeros((L,), jnp.int32)
        pltpu.sync_copy(tmpi, out_hbm.at[b])


_sc_call = pl.kernel(
    _sc_body,
    out_type=jax.ShapeDtypeStruct((B, L), jnp.int32),
    mesh=_MESH,
    compiler_params=pltpu.CompilerParams(
        needs_layout_passes=False, use_tc_tiling_on_sc=False
    ),
    scratch_types=[
        pltpu.VMEM((NV * L + 8,), jnp.float32),
        pltpu.VMEM_SHARED((NS * L,), jnp.float32),
        pltpu.VMEM_SHARED((NS * L,), jnp.int32),
        pltpu.VMEM((WPB * L,), jnp.float32),
        pltpu.VMEM((WPB * L,), jnp.int32),
        pltpu.VMEM((L,), jnp.float32),
        pltpu.VMEM((L,), jnp.int32),
    ],
)


_BLK = 128


def _tc_body(idx_ref, landms_ref, priors_ref, out_ref):
    b = pl.program_id(0)
    rel = lax.rem(idx_ref[b, 0], _BLK)
    contract = (((1,), (1,)), ((), ()))
    mask1 = (
        lax.broadcasted_iota(jnp.int32, (1, _BLK), 1) == rel
    ).astype(jnp.float32)
    bmask = (
        lax.broadcasted_iota(jnp.int32, (B, 1), 0) == b
    ).astype(jnp.float32)
    lv2 = jnp.sum(landms_ref[...] * bmask[None, :, :], axis=1)
    lv = lax.dot_general(
        mask1, lv2, contract, precision=lax.Precision.HIGHEST,
        preferred_element_type=jnp.float32,
    )
    pr = lax.dot_general(
        mask1, priors_ref[...], contract, precision=lax.Precision.HIGHEST,
        preferred_element_type=jnp.float32,
    )
    lv16 = jnp.concatenate([lv, jnp.zeros((1, L - 10), jnp.float32)], axis=1)
    kpar = lax.broadcasted_iota(jnp.int32, (1, L), 1) & 1
    pxy = jnp.where(kpar == 0, pr[:, 0:1], pr[:, 1:2])
    pwh = jnp.where(kpar == 0, pr[:, 2:3], pr[:, 3:4])
    out_ref[...] = (pxy + lv16 * VAR0 * pwh).reshape(1, 1, L)


_tc_call = pl.pallas_call(
    _tc_body,
    grid_spec=pltpu.PrefetchScalarGridSpec(
        num_scalar_prefetch=1,
        grid=(B,),
        in_specs=[
            pl.BlockSpec(
                (10, B, _BLK), lambda b, idx_ref: (0, 0, idx_ref[b, 0] // _BLK)
            ),
            pl.BlockSpec((B, _BLK), lambda b, idx_ref: (0, idx_ref[b, 0] // _BLK)),
        ],
        out_specs=pl.BlockSpec((1, 1, L), lambda b, idx_ref: (b, 0, 0)),
    ),
    out_shape=jax.ShapeDtypeStruct((B, 1, L), jnp.float32),
)


def kernel(loc, conf, landms, priors):
    del loc
    conf_f = conf.transpose(0, 2, 1).reshape(-1)
    idx_arr = _sc_call(conf_f)
    landms_t = landms.transpose(2, 0, 1)
    priors_t = priors.transpose(1, 0)
    out = _tc_call(idx_arr, landms_t, priors_t)
    return out[:, 0, :10]


# speedup vs baseline: 7.5142x; 1.1189x over previous
import jax
import jax.numpy as jnp
import numpy as np
from jax import lax
from jax.experimental import pallas as pl
from jax.experimental.pallas import tpu as pltpu
from jax.experimental.pallas import tpu_sc as plsc

B = 4
N = 16800
L = 16
NC = 1
NS = 16
WPB = 4
C = N // WPB
U = 4
NV = -(-C // L)
VAR0 = np.float32(0.1)
NEG_INF = np.float32(-np.inf)
IMAX = np.int32(2**31 - 1)

_MESH = plsc.VectorSubcoreMesh(
    core_axis_name="c", subcore_axis_name="s", num_cores=NC, num_subcores=NS
)


def _sc_body(scores_hbm, out_hbm, sbuf, mstage, istage, mload, iload, tmpf, tmpi):
    s = lax.axis_index("s")
    g = s // WPB
    w = s % WPB
    b = g
    base = w * C

    start = (2 * b + 1) * N + base
    a0 = (start // 8) * 8
    rem = start - a0
    pltpu.sync_copy(scores_hbm.at[pl.ds(a0, C + 4)], sbuf.at[pl.ds(0, C + 4)])

    lane = lax.iota(jnp.int32, L)
    lane_r = lane + rem

    def scan_vec(j, carry):
        run_max, run_idx = carry
        n = j * L + lane
        v = plsc.load_gather(sbuf, [j * L + lane_r])
        v = jnp.where(v > 0.0, v, NEG_INF)
        upd = v > run_max
        return jnp.where(upd, v, run_max), jnp.where(upd, base + n, run_idx)

    def step(i, chains):
        return tuple(scan_vec(i * U + k, chains[k]) for k in range(U))

    init = tuple(
        (jnp.full((L,), NEG_INF, jnp.float32), jnp.zeros((L,), jnp.int32))
        for _ in range(U)
    )
    nfull = (NV - 1) // U
    chains = lax.fori_loop(0, nfull, step, init)

    chains = tuple(
        scan_vec(nfull * U + k, chains[k]) if nfull * U + k < NV - 1 else chains[k]
        for k in range(U)
    )

    run_max, run_idx = chains[0]
    for m2, i2 in chains[1:]:
        upd = (m2 > run_max) | ((m2 == run_max) & (i2 < run_idx))
        run_max = jnp.where(upd, m2, run_max)
        run_idx = jnp.where(upd, i2, run_idx)

    n = (NV - 1) * L + lane
    v = plsc.load_gather(sbuf, [jnp.minimum(n, C - 1) + rem])
    v = jnp.where((v > 0.0) & (n < C), v, NEG_INF)
    upd = (v > run_max) | ((v == run_max) & (base + n < run_idx))
    run_max = jnp.where(upd, v, run_max)
    run_idx = jnp.where(upd, base + n, run_idx)

    tmpf[...] = run_max
    tmpi[...] = run_idx
    pltpu.sync_copy(tmpf, mstage.at[pl.ds(s * L, L)])
    pltpu.sync_copy(tmpi, istage.at[pl.ds(s * L, L)])
    plsc.subcore_barrier()

    @pl.when(w == 0)
    def _():
        pltpu.sync_copy(mstage.at[pl.ds(g * WPB * L, WPB * L)], mload)
        pltpu.sync_copy(istage.at[pl.ds(g * WPB * L, WPB * L)], iload)
        best_m = mload[pl.ds(0, L)]
        best_i = iload[pl.ds(0, L)]
        for k in range(1, WPB):
            m2 = mload[pl.ds(k * L, L)]
            i2 = iload[pl.ds(k * L, L)]
            upd = (m2 > best_m) | ((m2 == best_m) & (i2 < best_i))
            best_m = jnp.where(upd, m2, best_m)
            best_i = jnp.where(upd, i2, best_i)
        top = jnp.max(best_m, axis=0)
        cand = jnp.where(best_m == top, best_i, IMAX)
        tmpi[...] = jnp.min(cand, keepdims=True) + jnp.zeros((L,), jnp.int32)
        pltpu.sync_copy(tmpi, out_hbm.at[pl.ds(b * L, L)])


_sc_call = pl.kernel(
    _sc_body,
    out_type=jax.ShapeDtypeStruct((B * L,), jnp.int32),
    mesh=_MESH,
    compiler_params=pltpu.CompilerParams(
        needs_layout_passes=False, use_tc_tiling_on_sc=False
    ),
    scratch_types=[
        pltpu.VMEM((NV * L + 8,), jnp.float32),
        pltpu.VMEM_SHARED((NS * L,), jnp.float32),
        pltpu.VMEM_SHARED((NS * L,), jnp.int32),
        pltpu.VMEM((WPB * L,), jnp.float32),
        pltpu.VMEM((WPB * L,), jnp.int32),
        pltpu.VMEM((L,), jnp.float32),
        pltpu.VMEM((L,), jnp.int32),
    ],
)


_BLK = 128


def _tc_body(idx_ref, *refs):
    landms_refs = refs[:B]
    priors_refs = refs[B : 2 * B]
    out_ref = refs[2 * B]
    contract = (((1,), (1,)), ((), ()))
    kpar = lax.broadcasted_iota(jnp.int32, (1, L), 1) & 1
    nio = lax.broadcasted_iota(jnp.int32, (1, _BLK), 1)
    rows = []
    for b in range(B):
        rel = lax.rem(idx_ref[b * L], _BLK)
        mask1 = (nio == rel).astype(jnp.float32)
        lv = lax.dot_general(
            mask1, landms_refs[b][:, b, :], contract,
            precision=lax.Precision.HIGHEST,
            preferred_element_type=jnp.float32,
        )
        pr = lax.dot_general(
            mask1, priors_refs[b][...], contract,
            precision=lax.Precision.HIGHEST,
            preferred_element_type=jnp.float32,
        )
        lv16 = jnp.concatenate([lv, jnp.zeros((1, L - 10), jnp.float32)], axis=1)
        pxy = jnp.where(kpar == 0, pr[:, 0:1], pr[:, 1:2])
        pwh = jnp.where(kpar == 0, pr[:, 2:3], pr[:, 3:4])
        rows.append(pxy + lv16 * VAR0 * pwh)
    out_ref[...] = jnp.concatenate(rows, axis=0)


def _lm_spec(b):
    return pl.BlockSpec(
        (10, B, _BLK), lambda i, idx_ref: (0, 0, idx_ref[b * L] // _BLK)
    )


def _pr_spec(b):
    return pl.BlockSpec((B, _BLK), lambda i, idx_ref: (0, idx_ref[b * L] // _BLK))


_tc_call = pl.pallas_call(
    _tc_body,
    grid_spec=pltpu.PrefetchScalarGridSpec(
        num_scalar_prefetch=1,
        grid=(1,),
        in_specs=[_lm_spec(b) for b in range(B)] + [_pr_spec(b) for b in range(B)],
        out_specs=pl.BlockSpec((B, L), lambda i, idx_ref: (0, 0)),
    ),
    out_shape=jax.ShapeDtypeStruct((B, L), jnp.float32),
)


def kernel(loc, conf, landms, priors):
    del loc
    conf_f = conf.transpose(0, 2, 1).reshape(-1)
    idx_arr = _sc_call(conf_f)
    landms_t = landms.transpose(2, 0, 1)
    priors_t = priors.transpose(1, 0)
    out = _tc_call(idx_arr, *([landms_t] * B), *([priors_t] * B))
    return out[:, :10]


# speedup vs baseline: 7.5196x; 1.0007x over previous
import jax
import jax.numpy as jnp
import numpy as np
from jax import lax
from jax.experimental import pallas as pl
from jax.experimental.pallas import tpu as pltpu
from jax.experimental.pallas import tpu_sc as plsc

B = 4
N = 16800
L = 16
NC = 1
NS = 16
WPB = 4
C = N // WPB
U = 8
NV = -(-C // L)
VAR0 = np.float32(0.1)
NEG_INF = np.float32(-np.inf)
IMAX = np.int32(2**31 - 1)

_MESH = plsc.VectorSubcoreMesh(
    core_axis_name="c", subcore_axis_name="s", num_cores=NC, num_subcores=NS
)


def _sc_body(scores_hbm, out_hbm, sbuf, mstage, istage, mload, iload, tmpf, tmpi):
    s = lax.axis_index("s")
    g = s // WPB
    w = s % WPB
    b = g
    base = w * C

    start = (2 * b + 1) * N + base
    a0 = (start // 8) * 8
    rem = start - a0
    pltpu.sync_copy(scores_hbm.at[pl.ds(a0, C + 4)], sbuf.at[pl.ds(0, C + 4)])

    lane = lax.iota(jnp.int32, L)
    lane_r = lane + rem

    def scan_vec(j, carry):
        run_max, run_idx = carry
        n = j * L + lane
        v = plsc.load_gather(sbuf, [j * L + lane_r])
        v = jnp.where(v > 0.0, v, NEG_INF)
        upd = v > run_max
        return jnp.where(upd, v, run_max), jnp.where(upd, base + n, run_idx)

    def step(i, chains):
        return tuple(scan_vec(i * U + k, chains[k]) for k in range(U))

    init = tuple(
        (jnp.full((L,), NEG_INF, jnp.float32), jnp.zeros((L,), jnp.int32))
        for _ in range(U)
    )
    nfull = (NV - 1) // U
    chains = lax.fori_loop(0, nfull, step, init)

    chains = tuple(
        scan_vec(nfull * U + k, chains[k]) if nfull * U + k < NV - 1 else chains[k]
        for k in range(U)
    )

    run_max, run_idx = chains[0]
    for m2, i2 in chains[1:]:
        upd = (m2 > run_max) | ((m2 == run_max) & (i2 < run_idx))
        run_max = jnp.where(upd, m2, run_max)
        run_idx = jnp.where(upd, i2, run_idx)

    n = (NV - 1) * L + lane
    v = plsc.load_gather(sbuf, [jnp.minimum(n, C - 1) + rem])
    v = jnp.where((v > 0.0) & (n < C), v, NEG_INF)
    upd = (v > run_max) | ((v == run_max) & (base + n < run_idx))
    run_max = jnp.where(upd, v, run_max)
    run_idx = jnp.where(upd, base + n, run_idx)

    tmpf[...] = run_max
    tmpi[...] = run_idx
    pltpu.sync_copy(tmpf, mstage.at[pl.ds(s * L, L)])
    pltpu.sync_copy(tmpi, istage.at[pl.ds(s * L, L)])
    plsc.subcore_barrier()

    @pl.when(w == 0)
    def _():
        pltpu.sync_copy(mstage.at[pl.ds(g * WPB * L, WPB * L)], mload)
        pltpu.sync_copy(istage.at[pl.ds(g * WPB * L, WPB * L)], iload)
        best_m = mload[pl.ds(0, L)]
        best_i = iload[pl.ds(0, L)]
        for k in range(1, WPB):
            m2 = mload[pl.ds(k * L, L)]
            i2 = iload[pl.ds(k * L, L)]
            upd = (m2 > best_m) | ((m2 == best_m) & (i2 < best_i))
            best_m = jnp.where(upd, m2, best_m)
            best_i = jnp.where(upd, i2, best_i)
        top = jnp.max(best_m, axis=0)
        cand = jnp.where(best_m == top, best_i, IMAX)
        tmpi[...] = jnp.min(cand, keepdims=True) + jnp.zeros((L,), jnp.int32)
        pltpu.sync_copy(tmpi, out_hbm.at[pl.ds(b * L, L)])


_sc_call = pl.kernel(
    _sc_body,
    out_type=jax.ShapeDtypeStruct((B * L,), jnp.int32),
    mesh=_MESH,
    compiler_params=pltpu.CompilerParams(
        needs_layout_passes=False, use_tc_tiling_on_sc=False
    ),
    scratch_types=[
        pltpu.VMEM((NV * L + 8,), jnp.float32),
        pltpu.VMEM_SHARED((NS * L,), jnp.float32),
        pltpu.VMEM_SHARED((NS * L,), jnp.int32),
        pltpu.VMEM((WPB * L,), jnp.float32),
        pltpu.VMEM((WPB * L,), jnp.int32),
        pltpu.VMEM((L,), jnp.float32),
        pltpu.VMEM((L,), jnp.int32),
    ],
)


_BLK = 128


def _tc_body(idx_ref, *refs):
    landms_refs = refs[:B]
    priors_refs = refs[B : 2 * B]
    out_ref = refs[2 * B]
    contract = (((1,), (1,)), ((), ()))
    kpar = lax.broadcasted_iota(jnp.int32, (1, L), 1) & 1
    nio = lax.broadcasted_iota(jnp.int32, (1, _BLK), 1)
    rows = []
    for b in range(B):
        rel = lax.rem(idx_ref[b * L], _BLK)
        mask1 = (nio == rel).astype(jnp.float32)
        lv = lax.dot_general(
            mask1, landms_refs[b][:, b, :], contract,
            precision=lax.Precision.HIGHEST,
            preferred_element_type=jnp.float32,
        )
        pr = lax.dot_general(
            mask1, priors_refs[b][...], contract,
            precision=lax.Precision.HIGHEST,
            preferred_element_type=jnp.float32,
        )
        lv16 = jnp.concatenate([lv, jnp.zeros((1, L - 10), jnp.float32)], axis=1)
        pxy = jnp.where(kpar == 0, pr[:, 0:1], pr[:, 1:2])
        pwh = jnp.where(kpar == 0, pr[:, 2:3], pr[:, 3:4])
        rows.append(pxy + lv16 * VAR0 * pwh)
    out_ref[...] = jnp.concatenate(rows, axis=0)


def _lm_spec(b):
    return pl.BlockSpec(
        (10, B, _BLK), lambda i, idx_ref: (0, 0, idx_ref[b * L] // _BLK)
    )


def _pr_spec(b):
    return pl.BlockSpec((B, _BLK), lambda i, idx_ref: (0, idx_ref[b * L] // _BLK))


_tc_call = pl.pallas_call(
    _tc_body,
    grid_spec=pltpu.PrefetchScalarGridSpec(
        num_scalar_prefetch=1,
        grid=(1,),
        in_specs=[_lm_spec(b) for b in range(B)] + [_pr_spec(b) for b in range(B)],
        out_specs=pl.BlockSpec((B, L), lambda i, idx_ref: (0, 0)),
    ),
    out_shape=jax.ShapeDtypeStruct((B, L), jnp.float32),
)


def kernel(loc, conf, landms, priors):
    del loc
    conf_f = conf.transpose(0, 2, 1).reshape(-1)
    idx_arr = _sc_call(conf_f)
    landms_t = landms.transpose(2, 0, 1)
    priors_t = priors.transpose(1, 0)
    out = _tc_call(idx_arr, *([landms_t] * B), *([priors_t] * B))
    return out[:, :10]
